# baseline (speedup 1.0000x reference)
import jax
import jax.numpy as jnp
from jax.experimental import pallas as pl
from jax.experimental.pallas import tpu as pltpu

_R = 3
_M = 64
_N = 32768
_W = 32768
_MB = 16


def _copy_body(mask_ref, x_ref, o_ref):
    o_ref[...] = x_ref[...]


def _src_index_map(i, j, mask_ref):
    count = 0
    src = 0
    for row in range(_R):
        keep = 1 - mask_ref[row]
        hit = jnp.logical_and(count == i, keep == 1)
        src = jnp.where(hit, row, src)
        count = count + keep
    return (src, j, 0)


def kernel(x, bool_tensor):
    mask_i32 = bool_tensor.astype(jnp.int32)
    grid = (_R, _M // _MB)
    out = pl.pallas_call(
        _copy_body,
        grid_spec=pltpu.PrefetchScalarGridSpec(
            num_scalar_prefetch=1,
            grid=grid,
            in_specs=[
                pl.BlockSpec((1, _MB, _W), _src_index_map),
            ],
            out_specs=pl.BlockSpec((1, _MB, _W), lambda i, j, m: (i, j, 0)),
        ),
        out_shape=jax.ShapeDtypeStruct((_R, _M, _N), x.dtype),
    )(mask_i32, x)
    return out


# speedup vs baseline: 1.0980x; 1.0980x over previous
import jax
import jax.numpy as jnp
from jax.experimental import pallas as pl
from jax.experimental.pallas import tpu as pltpu

_R = 3
_M = 64
_N = 32768
_W = 32768
_MB = 32


def _copy_body(mask_ref, x_ref, o_ref):
    o_ref[...] = x_ref[...]


def _src_index_map(i, j, mask_ref):
    count = 0
    src = 0
    for row in range(_R):
        keep = 1 - mask_ref[row]
        hit = jnp.logical_and(count == i, keep == 1)
        src = jnp.where(hit, row, src)
        count = count + keep
    return (src, j, 0)


def kernel(x, bool_tensor):
    mask_i32 = bool_tensor.astype(jnp.int32)
    grid = (_R, _M // _MB)
    out = pl.pallas_call(
        _copy_body,
        grid_spec=pltpu.PrefetchScalarGridSpec(
            num_scalar_prefetch=1,
            grid=grid,
            in_specs=[
                pl.BlockSpec((1, _MB, _W), _src_index_map),
            ],
            out_specs=pl.BlockSpec((1, _MB, _W), lambda i, j, m: (i, j, 0)),
        ),
        out_shape=jax.ShapeDtypeStruct((_R, _M, _N), x.dtype),
    )(mask_i32, x)
    return out
